# Initial kernel scaffold; baseline (speedup 1.0000x reference)
import jax
import jax.numpy as jnp
from jax.experimental import pallas as pl


def kernel(src, variable_seq, emb_weight):
    raise NotImplementedError("write your pallas kernel here")


# speedup vs baseline: 1.9325x; 1.9325x over previous
import functools

import jax
import jax.numpy as jnp
from jax import lax
from jax.experimental import pallas as pl
from jax.experimental.pallas import tpu as pltpu
from jax.experimental.pallas import tpu_sc as plsc

VAR_LEN = 100000
EMBED = 64
B = 4096
L = 200
N = B * L

_info = plsc.get_sparse_core_info()
NC, NS = _info.num_cores, _info.num_subcores
NW = NC * NS
ROWS_PER_W = N // NW
CHUNK = 128
NCHUNK = ROWS_PER_W // CHUNK


def _sc_body(src_hbm, idx_hbm, table_hbm, out_hbm, idx_v, buf, sem):
    wid = lax.axis_index("s") * NC + lax.axis_index("c")
    wbase = wid * ROWS_PER_W

    def chunk_body(i, carry):
        base = wbase + i * CHUNK
        pltpu.sync_copy(idx_hbm.at[pl.ds(base, CHUNK)], idx_v)
        pltpu.sync_copy(src_hbm.at[pl.ds(base, CHUNK), :], buf)
        pltpu.async_copy(table_hbm.at[idx_v], buf, sem, add=True).wait()
        pltpu.sync_copy(buf, out_hbm.at[pl.ds(base, CHUNK), :])
        return carry

    lax.fori_loop(0, NCHUNK, chunk_body, 0)


@jax.jit
def _run(src_flat, idx_flat, emb_weight):
    mesh = plsc.VectorSubcoreMesh(core_axis_name="c", subcore_axis_name="s")
    f = functools.partial(
        pl.kernel,
        out_type=jax.ShapeDtypeStruct((N, EMBED), jnp.float32),
        mesh=mesh,
        scratch_types=[
            pltpu.VMEM((CHUNK,), jnp.int32),
            pltpu.VMEM((CHUNK, EMBED), jnp.float32),
            pltpu.SemaphoreType.DMA,
        ],
        compiler_params=pltpu.CompilerParams(use_tc_tiling_on_sc=False),
    )(_sc_body)
    return f(src_flat, idx_flat, emb_weight)


def kernel(src, variable_seq, emb_weight):
    src_flat = src.reshape(N, EMBED)
    idx_flat = variable_seq.reshape(N).astype(jnp.int32)
    out = _run(src_flat, idx_flat, emb_weight)
    return out.reshape(B, L, EMBED)


# speedup vs baseline: 2.3194x; 1.2002x over previous
import functools

import jax
import jax.numpy as jnp
from jax import lax
from jax.experimental import pallas as pl
from jax.experimental.pallas import tpu as pltpu
from jax.experimental.pallas import tpu_sc as plsc

VAR_LEN = 100000
EMBED = 64
B = 4096
L = 200
N = B * L

_info = plsc.get_sparse_core_info()
NC, NS = _info.num_cores, _info.num_subcores
NW = NC * NS
ROWS_PER_W = N // NW
IDX_W = 128
GPC = 2
R = IDX_W * GPC
NCH = ROWS_PER_W // R
IDX_ROWS_PER_W = ROWS_PER_W // IDX_W
NBUF = 4
T_OUTER = NCH // NBUF


def _sc_body(src_hbm, idx_hbm, table_hbm, out_hbm, idx_v, *rest):
    bufs = rest[0:NBUF]
    sem_a = rest[NBUF : 2 * NBUF]
    sem_b = rest[2 * NBUF : 3 * NBUF]
    sem_c = rest[3 * NBUF : 4 * NBUF]

    wid = lax.axis_index("s") * NC + lax.axis_index("c")
    wbase = wid * ROWS_PER_W
    cbase = wid * IDX_ROWS_PER_W

    pltpu.sync_copy(idx_hbm.at[pl.ds(cbase, IDX_ROWS_PER_W), :], idx_v)

    def load_desc(g, k):
        return pltpu.make_async_copy(
            src_hbm.at[pl.ds(wbase + g * R, R), :], bufs[k], sem_a[k]
        )

    def store_desc(g, k):
        return pltpu.make_async_copy(
            bufs[k], out_hbm.at[pl.ds(wbase + g * R, R), :], sem_c[k]
        )

    load_desc(0, 0).start()

    def outer(t, carry):
        for k in range(NBUF):
            g = t * NBUF + k
            load_desc(g, k).wait()
            descs = []
            for j in range(GPC):
                descs.append(
                    pltpu.async_copy(
                        table_hbm.at[idx_v.at[g * GPC + j]],
                        bufs[k].at[pl.ds(j * IDX_W, IDX_W), :],
                        sem_b[k],
                        add=True,
                    )
                )
            for d in descs:
                d.wait()
            store_desc(g, k).start()
            kn = (k + 1) % NBUF
            if k == NBUF - 1:
                store_desc(g + 1 - NBUF, kn).wait()

                @pl.when(t < T_OUTER - 1)
                def _():
                    load_desc(g + 1, kn).start()

            else:

                @pl.when(t > 0)
                def _():
                    store_desc(g + 1 - NBUF, kn).wait()

                load_desc(g + 1, kn).start()
        return carry

    lax.fori_loop(0, T_OUTER, outer, 0)

    for g in range(NCH - NBUF + 1, NCH):
        store_desc(g, g % NBUF).wait()


@jax.jit
def _run(src_flat, idx2d, emb_weight):
    mesh = plsc.VectorSubcoreMesh(core_axis_name="c", subcore_axis_name="s")
    scratch = [pltpu.VMEM((IDX_ROWS_PER_W, IDX_W), jnp.int32)]
    scratch += [pltpu.VMEM((R, EMBED), jnp.float32) for _ in range(NBUF)]
    scratch += [pltpu.SemaphoreType.DMA for _ in range(3 * NBUF)]
    f = functools.partial(
        pl.kernel,
        out_type=jax.ShapeDtypeStruct((N, EMBED), jnp.float32),
        mesh=mesh,
        scratch_types=scratch,
        compiler_params=pltpu.CompilerParams(use_tc_tiling_on_sc=False),
    )(_sc_body)
    return f(src_flat, idx2d, emb_weight)


def kernel(src, variable_seq, emb_weight):
    src_flat = src.reshape(N, EMBED)
    idx2d = variable_seq.reshape(N // IDX_W, IDX_W).astype(jnp.int32)
    out = _run(src_flat, idx2d, emb_weight)
    return out.reshape(B, L, EMBED)


# speedup vs baseline: 2.5045x; 1.0798x over previous
import functools

import jax
import jax.numpy as jnp
from jax import lax
from jax.experimental import pallas as pl
from jax.experimental.pallas import tpu as pltpu
from jax.experimental.pallas import tpu_sc as plsc

VAR_LEN = 100000
EMBED = 64
B = 4096
L = 200
N = B * L

_info = plsc.get_sparse_core_info()
NC, NS = _info.num_cores, _info.num_subcores
NW = NC * NS
ROWS_PER_W = N // NW
IDX_W = 128
GPC = 2
R = IDX_W * GPC
NCH = ROWS_PER_W // R
IDX_ROWS_PER_W = ROWS_PER_W // IDX_W
NBUF = 4
T_OUTER = NCH // NBUF


def _sc_body(src_hbm, idx_hbm, table_hbm, out_hbm, idx_v, *rest):
    bufs = rest[0:NBUF]
    sem_a = rest[NBUF : 2 * NBUF]
    sem_b = rest[2 * NBUF : 3 * NBUF]
    sem_c = rest[3 * NBUF : 4 * NBUF]

    wid = lax.axis_index("s") * NC + lax.axis_index("c")
    wbase = wid * ROWS_PER_W
    cbase = wid * IDX_ROWS_PER_W

    pltpu.sync_copy(idx_hbm.at[pl.ds(cbase, IDX_ROWS_PER_W), :], idx_v)

    def load_desc(g, k):
        return pltpu.make_async_copy(
            src_hbm.at[pl.ds(wbase + g * R, R), :], bufs[k], sem_a[k]
        )

    def store_desc(g, k):
        return pltpu.make_async_copy(
            bufs[k], out_hbm.at[pl.ds(wbase + g * R, R), :], sem_c[k]
        )

    def gather_descs(g, k):
        return [
            pltpu.make_async_copy(
                table_hbm.at[idx_v.at[g * GPC + j]],
                bufs[k].at[pl.ds(j * IDX_W, IDX_W), :],
                sem_b[k],
            )
            for j in range(GPC)
        ]

    def fire_gathers(g, k):
        for j in range(GPC):
            pltpu.async_copy(
                table_hbm.at[idx_v.at[g * GPC + j]],
                bufs[k].at[pl.ds(j * IDX_W, IDX_W), :],
                sem_b[k],
                add=True,
            )

    def drain_gathers(g, k):
        for d in gather_descs(g, k):
            d.wait()

    load_desc(0, 0).start()

    def outer(t, carry):
        for k in range(NBUF):
            g = t * NBUF + k
            kp = (k - 1) % NBUF
            kn = (k + 1) % NBUF
            load_desc(g, k).wait()
            fire_gathers(g, k)

            def retire():
                drain_gathers(g - 1, kp)
                store_desc(g - 1, kp).start()

            if k == 0:
                pl.when(t > 0)(retire)
            else:
                retire()

            def refill():
                store_desc(g + 1 - NBUF, kn).wait()

            if k == NBUF - 1:
                refill()
            else:
                pl.when(t > 0)(refill)

            @pl.when(g + 1 < NCH)
            def _():
                load_desc(g + 1, kn).start()

        return carry

    lax.fori_loop(0, T_OUTER, outer, 0)

    drain_gathers(NCH - 1, (NCH - 1) % NBUF)
    store_desc(NCH - 1, (NCH - 1) % NBUF).start()
    for g in range(NCH - NBUF + 1, NCH):
        store_desc(g, g % NBUF).wait()


@jax.jit
def _run(src_flat, idx2d, emb_weight):
    mesh = plsc.VectorSubcoreMesh(core_axis_name="c", subcore_axis_name="s")
    scratch = [pltpu.VMEM((IDX_ROWS_PER_W, IDX_W), jnp.int32)]
    scratch += [pltpu.VMEM((R, EMBED), jnp.float32) for _ in range(NBUF)]
    scratch += [pltpu.SemaphoreType.DMA for _ in range(3 * NBUF)]
    f = functools.partial(
        pl.kernel,
        out_type=jax.ShapeDtypeStruct((N, EMBED), jnp.float32),
        mesh=mesh,
        scratch_types=scratch,
        compiler_params=pltpu.CompilerParams(use_tc_tiling_on_sc=False),
    )(_sc_body)
    return f(src_flat, idx2d, emb_weight)


def kernel(src, variable_seq, emb_weight):
    src_flat = src.reshape(N, EMBED)
    idx2d = variable_seq.reshape(N // IDX_W, IDX_W).astype(jnp.int32)
    out = _run(src_flat, idx2d, emb_weight)
    return out.reshape(B, L, EMBED)


# speedup vs baseline: 4.1146x; 1.6429x over previous
import functools

import jax
import jax.numpy as jnp
from jax import lax
from jax.experimental import pallas as pl
from jax.experimental.pallas import tpu as pltpu
from jax.experimental.pallas import tpu_sc as plsc

VAR_LEN = 100000
EMBED = 64
B = 4096
L = 200

_info = plsc.get_sparse_core_info()
NC, NS, NL = _info.num_cores, _info.num_subcores, _info.num_lanes
NW = NC * NS
EPW = EMBED // NW
BT = B // 128
ET = EMBED // 8
LT = L // 8


def _sc_body(t_hbm, idx_hbm, tab_hbm, out_hbm, trow, idxb, sbuf,
             sem_i, sem_s, sem_o):
    cid = lax.axis_index("c")
    sid = lax.axis_index("s")
    wid = sid * NC + cid

    for p in range(EPW):
        e = wid * EPW + p
        e_t = lax.div(e, 8)
        e_8 = lax.rem(e, 8)

        def idx_load(l, k):
            return pltpu.make_async_copy(
                idx_hbm.at[lax.div(l, 8), :, lax.rem(l, 8), :],
                idxb[k], sem_i[k],
            )

        def src_load(l, k):
            return pltpu.make_async_copy(
                t_hbm.at[l * ET + e_t, :, e_8, :], sbuf[k], sem_s[k]
            )

        def out_store(l, k):
            return pltpu.make_async_copy(
                sbuf[k], out_hbm.at[l * ET + e_t, :, e_8, :], sem_o[k]
            )

        pltpu.sync_copy(tab_hbm.at[e], trow)

        idx_load(0, 0).start()
        src_load(0, 0).start()

        def outer(h, carry):
            for k in range(2):
                l = 2 * h + k
                kn = k ^ 1
                idx_load(l, k).wait()
                src_load(l, k).wait()

                @pl.when(l > 0)
                def _():
                    out_store(l - 1, kn).wait()

                @pl.when(l + 1 < L)
                def _():
                    idx_load(l + 1, kn).start()
                    src_load(l + 1, kn).start()

                def inner(r, c):
                    for u in range(128 // NL):
                        iv = idxb[k][r, pl.ds(u * NL, NL)]
                        g = plsc.load_gather(trow, [iv])
                        sbuf[k][r, pl.ds(u * NL, NL)] = (
                            sbuf[k][r, pl.ds(u * NL, NL)] + g
                        )
                    return c

                lax.fori_loop(0, BT, inner, 0)
                out_store(l, k).start()
            return carry

        lax.fori_loop(0, L // 2, outer, 0)
        out_store(L - 1, 1).wait()


@jax.jit
def _run(t4, idx4, tab_t):
    mesh = plsc.VectorSubcoreMesh(core_axis_name="c", subcore_axis_name="s")
    scratch = [
        pltpu.VMEM((VAR_LEN,), jnp.float32),
        [pltpu.VMEM((BT, 128), jnp.int32) for _ in range(2)],
        [pltpu.VMEM((BT, 128), jnp.float32) for _ in range(2)],
        [pltpu.SemaphoreType.DMA for _ in range(2)],
        [pltpu.SemaphoreType.DMA for _ in range(2)],
        [pltpu.SemaphoreType.DMA for _ in range(2)],
    ]
    f = functools.partial(
        pl.kernel,
        out_type=jax.ShapeDtypeStruct((L * ET, BT, 8, 128), jnp.float32),
        mesh=mesh,
        scratch_types=scratch,
        compiler_params=pltpu.CompilerParams(
            use_tc_tiling_on_sc=False, needs_layout_passes=False
        ),
    )(_sc_body)
    return f(t4, idx4, tab_t)


def kernel(src, variable_seq, emb_weight):
    t4 = (
        src.transpose(1, 2, 0)
        .reshape(L, ET, 8, BT, 128)
        .transpose(0, 1, 3, 2, 4)
        .reshape(L * ET, BT, 8, 128)
    )
    idx4 = (
        variable_seq.astype(jnp.int32)
        .transpose(1, 0)
        .reshape(LT, 8, BT, 128)
        .transpose(0, 2, 1, 3)
    )
    tab_t = emb_weight.transpose(1, 0)
    out4 = _run(t4, idx4, tab_t)
    return (
        out4.reshape(L, ET, BT, 8, 128)
        .transpose(0, 1, 3, 2, 4)
        .reshape(L, EMBED, B)
        .transpose(2, 0, 1)
    )


# speedup vs baseline: 6.2117x; 1.5097x over previous
import functools

import jax
import jax.numpy as jnp
from jax import lax
from jax.experimental import pallas as pl
from jax.experimental.pallas import tpu as pltpu
from jax.experimental.pallas import tpu_sc as plsc

VAR_LEN = 100000
EMBED = 64
B = 4096
L = 200

_info = plsc.get_sparse_core_info()
NC, NS, NL = _info.num_cores, _info.num_subcores, _info.num_lanes
NW = NC * NS
EPW = EMBED // NW
BT = B // 128
ET = EMBED // 8
LT = L // 8


def _sc_body(t_hbm, idx_hbm, tab_hbm, out_hbm, trow, idxb, sbuf,
             sem_i, sem_s, sem_o):
    cid = lax.axis_index("c")
    sid = lax.axis_index("s")
    wid = sid * NC + cid

    for p in range(EPW):
        e = wid * EPW + p
        e_t = lax.div(e, 8)
        e_8 = lax.rem(e, 8)

        def idx_load(l, k):
            return pltpu.make_async_copy(
                idx_hbm.at[lax.div(l, 8), :, lax.rem(l, 8), :],
                idxb[k], sem_i[k],
            )

        def src_load(l, k):
            return pltpu.make_async_copy(
                t_hbm.at[l * ET + e_t, :, e_8, :], sbuf[k], sem_s[k]
            )

        def out_store(l, k):
            return pltpu.make_async_copy(
                sbuf[k], out_hbm.at[l * ET + e_t, :, e_8, :], sem_o[k]
            )

        pltpu.sync_copy(tab_hbm.at[e], trow)

        idx_load(0, 0).start()
        src_load(0, 0).start()

        def outer(h, carry):
            for k in range(2):
                l = 2 * h + k
                kn = k ^ 1
                idx_load(l, k).wait()
                src_load(l, k).wait()

                @pl.when(l > 0)
                def _():
                    out_store(l - 1, kn).wait()

                @pl.when(l + 1 < L)
                def _():
                    idx_load(l + 1, kn).start()
                    src_load(l + 1, kn).start()

                @plsc.parallel_loop(0, BT, unroll=2)
                def _(r):
                    for u in range(128 // NL):
                        iv = idxb[k][r, pl.ds(u * NL, NL)]
                        g = plsc.load_gather(trow, [iv])
                        sbuf[k][r, pl.ds(u * NL, NL)] = (
                            sbuf[k][r, pl.ds(u * NL, NL)] + g
                        )
                out_store(l, k).start()
            return carry

        lax.fori_loop(0, L // 2, outer, 0)
        out_store(L - 1, 1).wait()


@jax.jit
def _run(t4, idx4, tab_t):
    mesh = plsc.VectorSubcoreMesh(core_axis_name="c", subcore_axis_name="s")
    scratch = [
        pltpu.VMEM((VAR_LEN,), jnp.float32),
        [pltpu.VMEM((BT, 128), jnp.int32) for _ in range(2)],
        [pltpu.VMEM((BT, 128), jnp.float32) for _ in range(2)],
        [pltpu.SemaphoreType.DMA for _ in range(2)],
        [pltpu.SemaphoreType.DMA for _ in range(2)],
        [pltpu.SemaphoreType.DMA for _ in range(2)],
    ]
    f = functools.partial(
        pl.kernel,
        out_type=jax.ShapeDtypeStruct((L * ET, BT, 8, 128), jnp.float32),
        mesh=mesh,
        scratch_types=scratch,
        compiler_params=pltpu.CompilerParams(
            use_tc_tiling_on_sc=False, needs_layout_passes=False
        ),
    )(_sc_body)
    return f(t4, idx4, tab_t)


def kernel(src, variable_seq, emb_weight):
    t4 = (
        src.transpose(1, 2, 0)
        .reshape(L, ET, 8, BT, 128)
        .transpose(0, 1, 3, 2, 4)
        .reshape(L * ET, BT, 8, 128)
    )
    idx4 = (
        variable_seq.astype(jnp.int32)
        .transpose(1, 0)
        .reshape(LT, 8, BT, 128)
        .transpose(0, 2, 1, 3)
    )
    tab_t = emb_weight.transpose(1, 0)
    out4 = _run(t4, idx4, tab_t)
    return (
        out4.reshape(L, ET, BT, 8, 128)
        .transpose(0, 1, 3, 2, 4)
        .reshape(L, EMBED, B)
        .transpose(2, 0, 1)
    )
